# Initial kernel scaffold; baseline (speedup 1.0000x reference)
import jax
import jax.numpy as jnp
from jax.experimental import pallas as pl


def kernel(inputs):
    raise NotImplementedError("write your pallas kernel here")


# speedup vs baseline: 5.3136x; 5.3136x over previous
import functools

import jax
import jax.numpy as jnp
from jax import lax
from jax.experimental import pallas as pl
from jax.experimental.pallas import tpu as pltpu
from jax.experimental.pallas import tpu_sc as plsc

_NUM_CORES = 2
_NUM_SUBCORES = 16
_NW = _NUM_CORES * _NUM_SUBCORES
_L = 16


def _row_logic(xs):
    zero_f = jnp.zeros((_L,), jnp.float32)
    zero_i = jnp.zeros((_L,), jnp.int32)
    one_i = jnp.ones((_L,), jnp.int32)

    def get_m(a, b, c):
        mx = jnp.maximum(a, jnp.maximum(b, c))
        e0 = (a == mx).astype(jnp.int32)
        e1 = (b == mx).astype(jnp.int32)
        e2 = (c == mx).astype(jnp.int32)
        cnt = e0 + e1 + e2
        return jnp.where(cnt == 1, e0 - e2, zero_i)

    up = xs[0:3]
    nn = xs[3:6]
    dn = xs[6:9]
    m_u = get_m(*up)
    m_n = get_m(*nn)
    m_d = get_m(*dn)
    calc = jnp.abs(m_n) * (m_u + m_d + m_n)
    s = jnp.sign(calc)
    keep_u = s == m_u
    keep_n = s == m_n
    keep_d = s == m_d
    up2 = [jnp.where(keep_u, v, zero_f) for v in up]
    nn2 = [jnp.where(keep_n, v, zero_f) for v in nn]
    dn2 = [jnp.where(keep_d, v, zero_f) for v in dn]
    idx = jnp.where(calc == 0, one_i, jnp.where(calc == 1, zero_i, 2 * one_i))

    def pick(g):
        return jnp.where(idx == 0, g[0], jnp.where(idx == 1, g[1], g[2]))

    val_u = pick(up2)
    val_n = pick(nn2)
    val_d = pick(dn2)
    w_u = (val_u >= val_n) & (val_u >= val_d)
    w_n = jnp.logical_not(w_u) & (val_n >= val_d)
    return [jnp.where(w_u, up2[j], jnp.where(w_n, nn2[j], dn2[j]))
            for j in range(3)]


def _make_sc_kernel(n_rows):
    rows_per_w = n_rows // _NW
    chunks = rows_per_w // _L
    in_words = rows_per_w * 9
    out_words = rows_per_w * 3
    mesh = plsc.VectorSubcoreMesh(core_axis_name="c", subcore_axis_name="s")

    @functools.partial(
        pl.kernel,
        out_type=jax.ShapeDtypeStruct((n_rows * 3,), jnp.float32),
        mesh=mesh,
        scratch_types=[
            pltpu.VMEM((in_words,), jnp.float32),
            pltpu.VMEM((out_words,), jnp.float32),
        ],
        compiler_params=pltpu.CompilerParams(needs_layout_passes=False),
    )
    def sc_kernel(x_hbm, out_hbm, x_v, out_v):
        wid = lax.axis_index("s") * _NUM_CORES + lax.axis_index("c")
        pltpu.sync_copy(x_hbm.at[pl.ds(wid * in_words, in_words)], x_v)

        lane = lax.broadcasted_iota(jnp.int32, (_L,), 0)
        lane9 = lane * 9
        lane3 = lane * 3

        def body(i, carry):
            rb9 = i * (_L * 9) + lane9
            rb3 = i * (_L * 3) + lane3
            xs = [plsc.load_gather(x_v, [rb9 + c]) for c in range(9)]
            outs = _row_logic(xs)
            for j in range(3):
                plsc.store_scatter(out_v, [rb3 + j], outs[j])
            return carry

        lax.fori_loop(0, chunks, body, 0)
        pltpu.sync_copy(out_v, out_hbm.at[pl.ds(wid * out_words, out_words)])

    return sc_kernel


def kernel(inputs):
    n_rows, n_feat = inputs.shape
    assert n_feat == 9 and n_rows % (_NW * _L) == 0
    out_flat = _make_sc_kernel(n_rows)(inputs.reshape(n_rows * 9))
    return out_flat.reshape(n_rows, 3)
